# baseline (speedup 1.0000x reference)
import functools

import jax
import jax.numpy as jnp
from jax import lax
from jax.experimental import pallas as pl
from jax.experimental.pallas import tpu as pltpu
from jax.experimental.pallas import tpu_sc as plsc

_BATCH = 4
_SEQ = 8192
_HIDDEN = 1024
_NC = 2
_NS = 16
_NW = _NC * _NS
_ROWS_PER_W = _SEQ // _NW
_CHUNK = 32
_NCHUNK = _ROWS_PER_W // _CHUNK


@functools.partial(
    pl.kernel,
    mesh=plsc.VectorSubcoreMesh(core_axis_name="c", subcore_axis_name="s"),
    out_type=jax.ShapeDtypeStruct((_BATCH, _SEQ, _HIDDEN), jnp.float32),
    scratch_types=[
        pltpu.SemaphoreType.DMA,
    ],
)
def _broadcast_table(table_hbm, out_hbm, ssem):
    wid = lax.axis_index("s") * _NC + lax.axis_index("c")
    base = wid * _ROWS_PER_W

    copies = [
        pltpu.make_async_copy(
            table_hbm.at[pl.ds(base, _ROWS_PER_W), :],
            out_hbm.at[b, pl.ds(base, _ROWS_PER_W), :],
            ssem)
        for b in range(_BATCH)
    ]
    for c in copies:
        c.start()
    for c in copies:
        c.wait()


def kernel(x, table):
    del x
    return _broadcast_table(table)


# speedup vs baseline: 54.4566x; 54.4566x over previous
import functools

import jax
import jax.numpy as jnp
from jax import lax
from jax.experimental import pallas as pl
from jax.experimental.pallas import tpu as pltpu
from jax.experimental.pallas import tpu_sc as plsc

_BATCH = 4
_SEQ = 8192
_HIDDEN = 1024
_NC = 2
_NS = 16
_NW = _NC * _NS
_ROWS_PER_W = _SEQ // _NW
_CHUNK = 32
_NCHUNK = _ROWS_PER_W // _CHUNK


@functools.partial(
    pl.kernel,
    mesh=plsc.VectorSubcoreMesh(core_axis_name="c", subcore_axis_name="s"),
    out_type=jax.ShapeDtypeStruct((_BATCH, _SEQ, _HIDDEN), jnp.float32),
    scratch_types=[
        pltpu.VMEM((_CHUNK, _HIDDEN), jnp.float32),
        pltpu.VMEM((_CHUNK, _HIDDEN), jnp.float32),
        pltpu.VMEM((_CHUNK, _HIDDEN), jnp.float32),
        pltpu.SemaphoreType.DMA,
        pltpu.SemaphoreType.DMA,
        pltpu.SemaphoreType.DMA,
        pltpu.SemaphoreType.DMA,
        pltpu.SemaphoreType.DMA,
        pltpu.SemaphoreType.DMA,
    ],
)
def _broadcast_table(table_hbm, out_hbm, buf0, buf1, buf2,
                     lsem0, lsem1, lsem2, ssem0, ssem1, ssem2):
    wid = lax.axis_index("s") * _NC + lax.axis_index("c")
    base = wid * _ROWS_PER_W
    bufs = (buf0, buf1, buf2)
    lsems = (lsem0, lsem1, lsem2)
    ssems = (ssem0, ssem1, ssem2)

    def load(i):
        r0 = base + i * _CHUNK
        return pltpu.make_async_copy(
            table_hbm.at[pl.ds(r0, _CHUNK), :], bufs[i % 3], lsems[i % 3])

    def stores(i):
        r0 = base + i * _CHUNK
        return [
            pltpu.make_async_copy(
                bufs[i % 3], out_hbm.at[b, pl.ds(r0, _CHUNK), :], ssems[i % 3])
            for b in range(_BATCH)
        ]

    load(0).start()
    load(1).start()
    for i in range(_NCHUNK):
        load(i).wait()
        for s in stores(i):
            s.start()
        if i + 2 < _NCHUNK:
            if i >= 1:
                for s in stores(i - 1):
                    s.wait()
            load(i + 2).start()
    for i in (_NCHUNK - 3, _NCHUNK - 2, _NCHUNK - 1):
        for s in stores(i):
            s.wait()


def kernel(x, table):
    del x
    return _broadcast_table(table)
